# baseline (speedup 1.0000x reference)
import functools

import jax
import jax.numpy as jnp
from jax import lax
from jax.experimental import pallas as pl
from jax.experimental.pallas import tpu as pltpu
from jax.experimental.pallas import tpu_sc as plsc

_NC, _NS = 2, 16
_NW = _NC * _NS
_SPLIT = 8
_NBUF = 4


def kernel(r_idx, qkv):
    n, p3, w3, c = qkv.shape
    topk = r_idx.shape[-1]
    rows = n * p3
    out_rows = rows * topk
    d = w3 * c
    sd = d // _SPLIT
    rows_pt = out_rows // _NW
    sub_pt = rows_pt * _SPLIT
    chunks = rows_pt

    table = qkv.reshape(rows * _SPLIT, sd)

    rif = r_idx.astype(jnp.int32).reshape(-1)
    gidx = rif + (jnp.arange(out_rows, dtype=jnp.int32) // (p3 * topk)) * p3
    sidx = (gidx[:, None] * _SPLIT + jnp.arange(_SPLIT, dtype=jnp.int32)).reshape(
        _NW, sub_pt)
    sidx2d = jnp.pad(sidx, ((0, 0), (0, 512 - sub_pt)))

    mesh = plsc.VectorSubcoreMesh(
        core_axis_name="c", subcore_axis_name="s",
        num_cores=_NC, num_subcores=_NS,
    )

    @functools.partial(
        pl.kernel,
        out_type=jax.ShapeDtypeStruct((out_rows * _SPLIT, sd), jnp.float32),
        mesh=mesh,
        scratch_types=[
            pltpu.VMEM((512,), jnp.int32),
            pltpu.VMEM((_NBUF, _SPLIT, sd), jnp.float32),
            [pltpu.SemaphoreType.DMA] * _NBUF,
            [pltpu.SemaphoreType.DMA] * _NBUF,
        ],
    )
    def sc_gather(sidx_hbm, table_hbm, out_hbm, sidx_v, buf, gsems, ssems):
        wid = lax.axis_index("s") * _NC + lax.axis_index("c")
        pltpu.sync_copy(sidx_hbm.at[wid], sidx_v)

        out_base = wid * sub_pt
        gath = [None] * _NBUF
        scat = [None] * _NBUF
        for g in range(chunks):
            b = g % _NBUF
            if g >= _NBUF:
                scat[b].wait()
            gath[b] = pltpu.async_copy(
                table_hbm.at[sidx_v.at[pl.ds(g * _SPLIT, _SPLIT)]],
                buf.at[b], gsems[b])
            if g >= 1:
                pb = (g - 1) % _NBUF
                gath[pb].wait()
                scat[pb] = pltpu.async_copy(
                    buf.at[pb],
                    out_hbm.at[pl.ds(out_base + (g - 1) * _SPLIT, _SPLIT)],
                    ssems[pb])
        lb = (chunks - 1) % _NBUF
        gath[lb].wait()
        scat[lb] = pltpu.async_copy(
            buf.at[lb],
            out_hbm.at[pl.ds(out_base + (chunks - 1) * _SPLIT, _SPLIT)],
            ssems[lb])
        for cch in range(max(0, chunks - _NBUF), chunks):
            scat[cch % _NBUF].wait()

    out = sc_gather(sidx2d, table)
    return out.reshape(n, p3, topk, w3, c)


# speedup vs baseline: 2.5092x; 2.5092x over previous
import functools

import jax
import jax.numpy as jnp
from jax import lax
from jax.experimental import pallas as pl
from jax.experimental.pallas import tpu as pltpu
from jax.experimental.pallas import tpu_sc as plsc

_NC, _NS = 2, 16
_NW = _NC * _NS
_SPLIT = 8
_NBUF = 4


def kernel(r_idx, qkv):
    n, p3, w3, c = qkv.shape
    topk = r_idx.shape[-1]
    rows = n * p3
    out_rows = rows * topk
    d = w3 * c
    sd = d // _SPLIT
    rows_pt = out_rows // _NW
    sub_pt = rows_pt * _SPLIT
    chunks = rows_pt

    table = qkv.reshape(rows * _SPLIT, w3 // _SPLIT, c)

    rif = r_idx.astype(jnp.int32).reshape(-1)
    gidx = rif + (jnp.arange(out_rows, dtype=jnp.int32) // (p3 * topk)) * p3
    sidx = (gidx[:, None] * _SPLIT + jnp.arange(_SPLIT, dtype=jnp.int32)).reshape(
        _NW, sub_pt)
    sidx1d = jnp.pad(sidx, ((0, 0), (0, 512 - sub_pt))).reshape(-1)

    mesh = plsc.VectorSubcoreMesh(
        core_axis_name="c", subcore_axis_name="s",
        num_cores=_NC, num_subcores=_NS,
    )

    @functools.partial(
        pl.kernel,
        out_type=jax.ShapeDtypeStruct(
            (out_rows * _SPLIT, w3 // _SPLIT, c), jnp.float32),
        mesh=mesh,
        compiler_params=pltpu.CompilerParams(use_tc_tiling_on_sc=True),
        scratch_types=[
            pltpu.VMEM((512,), jnp.int32),
            pltpu.VMEM((_NBUF, _SPLIT, w3 // _SPLIT, c), jnp.float32),
            [pltpu.SemaphoreType.DMA] * _NBUF,
            [pltpu.SemaphoreType.DMA] * _NBUF,
        ],
    )
    def sc_gather(sidx_hbm, table_hbm, out_hbm, sidx_v, buf, gsems, ssems):
        wid = lax.axis_index("s") * _NC + lax.axis_index("c")
        pltpu.sync_copy(sidx_hbm.at[pl.ds(wid * 512, 512)], sidx_v)

        out_base = wid * sub_pt
        gath = [None] * _NBUF
        scat = [None] * _NBUF
        for g in range(chunks):
            b = g % _NBUF
            if g >= _NBUF:
                scat[b].wait()
            gath[b] = pltpu.async_copy(
                table_hbm.at[sidx_v.at[pl.ds(g * _SPLIT, _SPLIT)]],
                buf.at[b], gsems[b])
            if g >= 1:
                pb = (g - 1) % _NBUF
                gath[pb].wait()
                scat[pb] = pltpu.async_copy(
                    buf.at[pb],
                    out_hbm.at[pl.ds(out_base + (g - 1) * _SPLIT, _SPLIT)],
                    ssems[pb])
        lb = (chunks - 1) % _NBUF
        gath[lb].wait()
        scat[lb] = pltpu.async_copy(
            buf.at[lb],
            out_hbm.at[pl.ds(out_base + (chunks - 1) * _SPLIT, _SPLIT)],
            ssems[lb])
        for cch in range(max(0, chunks - _NBUF), chunks):
            scat[cch % _NBUF].wait()

    out = sc_gather(sidx1d, table)
    return out.reshape(n, p3, topk, w3, c)


# speedup vs baseline: 2.5240x; 1.0059x over previous
import functools

import jax
import jax.numpy as jnp
from jax import lax
from jax.experimental import pallas as pl
from jax.experimental.pallas import tpu as pltpu
from jax.experimental.pallas import tpu_sc as plsc

_NC, _NS = 2, 16
_NW = _NC * _NS
_SPLIT = 8
_CH = 16
_NBUF = 2


def kernel(r_idx, qkv):
    n, p3, w3, c = qkv.shape
    topk = r_idx.shape[-1]
    rows = n * p3
    out_rows = rows * topk
    d = w3 * c
    sd = d // _SPLIT
    rows_pt = out_rows // _NW
    sub_pt = rows_pt * _SPLIT
    chunks = rows_pt

    table = qkv.reshape(rows * _SPLIT, w3 // _SPLIT, c)

    rif = r_idx.astype(jnp.int32).reshape(-1)
    gidx = rif + (jnp.arange(out_rows, dtype=jnp.int32) // (p3 * topk)) * p3
    sidx = (gidx[:, None] * _SPLIT + jnp.arange(_SPLIT, dtype=jnp.int32)).reshape(
        _NW, sub_pt)
    sidx1d = jnp.pad(sidx, ((0, 0), (0, 512 - sub_pt))).reshape(-1)

    mesh = plsc.VectorSubcoreMesh(
        core_axis_name="c", subcore_axis_name="s",
        num_cores=_NC, num_subcores=_NS,
    )

    @functools.partial(
        pl.kernel,
        out_type=jax.ShapeDtypeStruct(
            (out_rows * _SPLIT, w3 // _SPLIT, c), jnp.float32),
        mesh=mesh,
        compiler_params=pltpu.CompilerParams(use_tc_tiling_on_sc=True),
        scratch_types=[
            pltpu.VMEM((512,), jnp.int32),
            pltpu.VMEM((_NBUF, _CH, w3 // _SPLIT, c), jnp.float32),
            [pltpu.SemaphoreType.DMA] * _NBUF,
            [pltpu.SemaphoreType.DMA] * _NBUF,
        ],
    )
    def sc_gather(sidx_hbm, table_hbm, out_hbm, sidx_v, buf, gsems, ssems):
        wid = lax.axis_index("s") * _NC + lax.axis_index("c")
        pltpu.sync_copy(sidx_hbm.at[pl.ds(wid * 512, 512)], sidx_v)

        out_base = wid * sub_pt
        spans = [(s0, min(_CH, sub_pt - s0)) for s0 in range(0, sub_pt, _CH)]
        nch = len(spans)
        gath = [None] * _NBUF
        scat = [None] * _NBUF
        for g, (s0, sz) in enumerate(spans):
            b = g % _NBUF
            if g >= _NBUF:
                scat[b].wait()
            gath[b] = pltpu.async_copy(
                table_hbm.at[sidx_v.at[pl.ds(s0, sz)]],
                buf.at[b, pl.ds(0, sz)], gsems[b])
            if g >= 1:
                pb = (g - 1) % _NBUF
                p0, psz = spans[g - 1]
                gath[pb].wait()
                scat[pb] = pltpu.async_copy(
                    buf.at[pb, pl.ds(0, psz)],
                    out_hbm.at[pl.ds(out_base + p0, psz)],
                    ssems[pb])
        lb = (nch - 1) % _NBUF
        l0, lsz = spans[-1]
        gath[lb].wait()
        scat[lb] = pltpu.async_copy(
            buf.at[lb, pl.ds(0, lsz)],
            out_hbm.at[pl.ds(out_base + l0, lsz)],
            ssems[lb])
        for cch in range(max(0, nch - _NBUF), nch):
            scat[cch % _NBUF].wait()

    out = sc_gather(sidx1d, table)
    return out.reshape(n, p3, topk, w3, c)
